# baseline (speedup 1.0000x reference)
import functools

import jax
import jax.numpy as jnp
from jax import lax
from jax.experimental import pallas as pl
from jax.experimental.pallas import tpu as pltpu
from jax.experimental.pallas import tpu_sc as plsc

VOCAB = 1000000
EMBED = 64
BATCH = 16384
HIST = 50

_NBLK = VOCAB // 128
_TAIL = VOCAB - _NBLK * 128
_RM_ROWS = VOCAB // 2


def _iota16():
    return lax.iota(jnp.int32, 16)


@functools.cache
def _build_kernels():
    info = plsc.get_sparse_core_info()
    nc, ns = info.num_cores, info.num_subcores
    nw = nc * ns
    mesh = plsc.VectorSubcoreMesh(core_axis_name="c", subcore_axis_name="s")
    tiled = pltpu.CompilerParams(
        use_tc_tiling_on_sc=True, needs_layout_passes=False
    )
    n_iter_a = -(-_NBLK // nw)

    @functools.partial(
        pl.kernel,
        mesh=mesh,
        out_type=jax.ShapeDtypeStruct((_RM_ROWS, 128), jnp.float32),
        scratch_types=[
            pltpu.VMEM((2, EMBED, 128), jnp.float32),
            pltpu.VMEM((EMBED, 128), jnp.float32),
            pltpu.SemaphoreType.DMA,
            pltpu.SemaphoreType.DMA,
        ],
        compiler_params=tiled,
    )
    def transpose_kernel(tT_hbm, rm_hbm, bufs, obuf, sem0, sem1):
        sems = (sem0, sem1)
        wid = lax.axis_index("s") * nc + lax.axis_index("c")
        i16 = _iota16()
        i0s = [i16 + 16 * (s % 4) for s in range(8)]

        def blk(t):
            return wid + nw * t

        def fire(t, d):
            @pl.when(blk(t) < _NBLK)
            def _():
                pltpu.async_copy(
                    tT_hbm.at[:, pl.ds(blk(t) * 128, 128)], bufs.at[d], sems[d]
                )

        def transpose_block(d, n_rows):
            @pl.loop(0, n_rows)
            def _(p):
                for s in range(8):
                    vals = plsc.load_gather(
                        bufs.at[d], [i0s[s], jnp.full((16,), 0, jnp.int32) + 2 * p + (s // 4)]
                    )
                    obuf[p, pl.ds(16 * s, 16)] = vals

        for d in range(2):
            fire(d, d)

        @pl.loop(0, n_iter_a + 1, step=2)
        def _(g):
            for d in range(2):
                t = g + d

                @pl.when(blk(t) < _NBLK)
                def _():
                    pltpu.make_async_copy(
                        tT_hbm.at[:, pl.ds(blk(t) * 128, 128)], bufs.at[d], sems[d]
                    ).wait()
                    transpose_block(d, 64)
                    pltpu.sync_copy(obuf, rm_hbm.at[pl.ds(blk(t) * 64, 64)])

                fire(t + 2, d)

    n_c_per_w = (BATCH // 128) // nw

    @functools.partial(
        pl.kernel,
        mesh=mesh,
        out_type=jax.ShapeDtypeStruct((HIST, EMBED, BATCH), jnp.float32),
        scratch_types=[
            pltpu.VMEM((HIST, 128), jnp.int32),
            pltpu.VMEM((2, 128), jnp.int32),
            pltpu.VMEM((2, 128, 128), jnp.float32),
            pltpu.VMEM((EMBED, 128), jnp.float32),
            pltpu.VMEM((_TAIL * EMBED,), jnp.float32),
            pltpu.SemaphoreType.DMA,
            pltpu.SemaphoreType.DMA,
        ],
        compiler_params=tiled,
    )
    def gather_kernel(
        xT_hbm, rm_hbm, tails_hbm, out_hbm, xbuf, idxp, rows, obuf, tails_v,
        sem0, sem1,
    ):
        sems = (sem0, sem1)
        wid = lax.axis_index("s") * nc + lax.axis_index("c")
        i16 = _iota16()
        i0s = [i16 + 16 * s for s in range(8)]
        kfull = (HIST // 8) * 8
        bound = _NBLK * 128

        pltpu.sync_copy(tails_hbm, tails_v)

        for cc in range(n_c_per_w):
            c = wid * n_c_per_w + cc
            pltpu.sync_copy(
                xT_hbm.at[pl.ds(0, kfull), pl.ds(c * 128, 128)],
                xbuf.at[pl.ds(0, kfull)],
            )
            pltpu.sync_copy(
                xT_hbm.at[pl.ds(kfull, HIST - kfull), pl.ds(c * 128, 128)],
                xbuf.at[pl.ds(kfull, HIST - kfull)],
            )

            def stage(k, d):
                @pl.when(k < HIST)
                def _():
                    for s in range(8):
                        v = xbuf[k, pl.ds(16 * s, 16)]
                        idxp[d, pl.ds(16 * s, 16)] = lax.shift_right_logical(v, 1)
                    pltpu.async_copy(rm_hbm.at[idxp.at[d]], rows.at[d], sems[d])

            for d in range(2):
                stage(jnp.int32(d), d)

            @pl.loop(0, HIST, step=2)
            def _(g):
                for d in range(2):
                    k = g + d
                    pltpu.make_async_copy(
                        rm_hbm.at[idxp.at[d]], rows.at[d], sems[d]
                    ).wait()
                    vs = [xbuf[k, pl.ds(16 * s, 16)] for s in range(8)]
                    pvs = [
                        lax.shift_left(lax.bitwise_and(v, 1), 6) for v in vs
                    ]
                    hits = sum(
                        jnp.where(v >= bound, 1, 0).astype(jnp.int32)
                        for v in vs
                    )
                    n_tail = lax.reduce_max(hits, axes=(0,))

                    @pl.when(n_tail == 0)
                    def _():
                        @pl.loop(0, EMBED)
                        def _(j):
                            for s in range(8):
                                vals = plsc.load_gather(
                                    rows.at[d], [i0s[s], pvs[s] + j]
                                )
                                obuf[j, pl.ds(16 * s, 16)] = vals

                    @pl.when(n_tail > 0)
                    def _():
                        masks = [v >= bound for v in vs]
                        tbases = [
                            jnp.maximum(v - bound, 0) * EMBED for v in vs
                        ]

                        @pl.loop(0, EMBED)
                        def _(j):
                            for s in range(8):
                                vals = plsc.load_gather(
                                    rows.at[d], [i0s[s], pvs[s] + j]
                                )
                                tvals = plsc.load_gather(
                                    tails_v, [tbases[s] + j]
                                )
                                obuf[j, pl.ds(16 * s, 16)] = jnp.where(
                                    masks[s], tvals, vals
                                )

                    stage(k + 2, d)
                    pltpu.sync_copy(
                        obuf, out_hbm.at[k, :, pl.ds(c * 128, 128)]
                    )

    return transpose_kernel, gather_kernel


@jax.jit
def kernel(x, table):
    t_kernel, g_kernel = _build_kernels()
    rm = t_kernel(table.T)
    tails = table[_NBLK * 128:].reshape(-1)
    out_p = g_kernel(x.T.astype(jnp.int32), rm, tails)
    return jnp.transpose(out_p, (2, 0, 1))
